# baseline (speedup 1.0000x reference)
import functools

import jax
import jax.numpy as jnp
from jax import lax
from jax.experimental import pallas as pl
from jax.experimental.pallas import tpu as pltpu
from jax.experimental.pallas import tpu_sc as plsc

N_NODES = 10000
N_PAD = 10240
E = 320000
D = 128
NG = 64
NC = 2
NS = 16
NW = NC * NS
CHUNK = 128
EPT = 80
E_PAD = NW * EPT * CHUNK
NCHUNKS = E_PAD // CHUNK
NBUF = 4
RPT = N_PAD // NS
BLK = 512
GRID = N_PAD // BLK


@functools.lru_cache(maxsize=None)
def _sc_agg():
    mesh = plsc.VectorSubcoreMesh(
        core_axis_name="c", subcore_axis_name="s",
        num_cores=NC, num_subcores=NS)

    @functools.partial(
        pl.kernel,
        out_type=(
            jax.ShapeDtypeStruct((N_PAD, D), jnp.float32),
            jax.ShapeDtypeStruct((N_PAD, D), jnp.float32),
        ),
        mesh=mesh,
        scratch_types=[
            pltpu.VMEM((CHUNK,), jnp.int32),
            pltpu.VMEM((CHUNK,), jnp.int32),
            [pltpu.VMEM((CHUNK, D), jnp.float32)] * NBUF,
            [pltpu.SemaphoreType.DMA] * NBUF,
            [pltpu.SemaphoreType.DMA] * NBUF,
            pltpu.VMEM_SHARED((N_PAD, D), jnp.float32),
        ],
    )
    def _agg(table, src2, dst2, zeros, out0, out1,
             src_v, dst_v, rows, gsem, ssem, acc):
        c = lax.axis_index("c")
        s = lax.axis_index("s")
        r0 = pl.multiple_of(s * RPT, RPT)
        pltpu.sync_copy(zeros, acc.at[pl.ds(r0, RPT)])
        wid = c * NS + s
        ebase = wid * (EPT * CHUNK)
        plsc.subcore_barrier()

        def body(g, carry):
            off = pl.multiple_of(ebase + g * CHUNK, CHUNK)
            pltpu.sync_copy(src2.at[pl.ds(off, CHUNK)], src_v)
            pltpu.sync_copy(dst2.at[pl.ds(off, CHUNK)], dst_v)
            pltpu.async_copy(table.at[src_v], rows[0], gsem[0]).wait()
            pltpu.sync_copy(rows[0], acc.at[dst_v], add=True)
            return carry

        lax.fori_loop(0, EPT, body, 0)
        plsc.subcore_barrier()

        @pl.when(c == 0)
        def _():
            pltpu.sync_copy(acc.at[pl.ds(r0, RPT)], out0.at[pl.ds(r0, RPT)])

        @pl.when(c == 1)
        def _():
            pltpu.sync_copy(acc.at[pl.ds(r0, RPT)], out1.at[pl.ds(r0, RPT)])

    return _agg


def _hidden(p0, p1, ea0, ea1, wn, we, bias):
    agg = jnp.dot(p0[...] + p1[...], wn[...],
                  preferred_element_type=jnp.float32)
    e_blk = ea0[...] + ea1[...]
    esum = e_blk[:, :32]
    deg = e_blk[:, 32:33]
    aggf = agg + jnp.dot(esum, we[...], preferred_element_type=jnp.float32)
    return jnp.maximum(aggf / jnp.maximum(deg, 1.0) + bias[...], 0.0)


def _boundary_body(p0, p1, ea0, ea1, wn, we, bias, o_ref):
    o_ref[...] = _hidden(p0, p1, ea0, ea1, wn, we, bias)


_NODE_SPECS = [
    pl.BlockSpec((BLK, D), lambda i: (i, 0)),
    pl.BlockSpec((BLK, D), lambda i: (i, 0)),
    pl.BlockSpec((BLK, D), lambda i: (i, 0)),
    pl.BlockSpec((BLK, D), lambda i: (i, 0)),
    pl.BlockSpec((D, D), lambda i: (0, 0)),
    pl.BlockSpec((32, D), lambda i: (0, 0)),
    pl.BlockSpec((1, D), lambda i: (0, 0)),
]


def _boundary(p0, p1, ea0, ea1, wn, we, bias):
    return pl.pallas_call(
        _boundary_body,
        out_shape=jax.ShapeDtypeStruct((N_PAD, D), jnp.float32),
        grid=(GRID,),
        in_specs=_NODE_SPECS,
        out_specs=pl.BlockSpec((BLK, D), lambda i: (i, 0)),
    )(p0, p1, ea0, ea1, wn, we, bias)


def _final_body(p0, p1, ea0, ea1, wn, we, bias, oh_ref, o_ref):
    h = _hidden(p0, p1, ea0, ea1, wn, we, bias)

    @pl.when(pl.program_id(0) == 0)
    def _():
        o_ref[...] = jnp.zeros_like(o_ref)

    oh = oh_ref[...]
    rows = [jnp.max(h * oh[:, g:g + 1], axis=0) for g in range(NG)]
    o_ref[...] = jnp.maximum(o_ref[...], jnp.stack(rows, axis=0))


def _final(p0, p1, ea0, ea1, wn, we, bias, oh):
    return pl.pallas_call(
        _final_body,
        out_shape=jax.ShapeDtypeStruct((NG, D), jnp.float32),
        grid=(GRID,),
        in_specs=_NODE_SPECS + [pl.BlockSpec((BLK, NG), lambda i: (i, 0))],
        out_specs=pl.BlockSpec((NG, D), lambda i: (0, 0)),
    )(p0, p1, ea0, ea1, wn, we, bias, oh)


def kernel(x, edge_attr, edge_index, batch, embed, edge_embed, W_node, W_edge, b):
    x = x.astype(jnp.int32)
    ea = edge_attr.astype(jnp.int32)
    src = edge_index[0].astype(jnp.int32)
    dst = edge_index[1].astype(jnp.int32)
    bt = batch.astype(jnp.int32)

    pad_e = E_PAD - E
    xs = x[src]
    xs_pad = jnp.concatenate([xs, jnp.zeros((pad_e,), jnp.int32)])
    src_pad = jnp.concatenate([src, jnp.zeros((pad_e,), jnp.int32)])
    dst_pad = jnp.concatenate([dst, jnp.full((pad_e,), N_NODES, jnp.int32)])
    ea_pad = jnp.concatenate([ea, jnp.full((pad_e,), 200, jnp.int32)])
    xs2, src2, dst2, ea2 = xs_pad, src_pad, dst_pad, ea_pad

    aug = jnp.zeros((208, D), jnp.float32)
    aug = aug.at[:200, :32].set(edge_embed).at[:200, 32].set(1.0)

    zeros128 = jnp.zeros((RPT, D), jnp.float32)
    oh = jnp.concatenate(
        [jax.nn.one_hot(bt, NG, dtype=jnp.float32),
         jnp.zeros((N_PAD - N_NODES, NG), jnp.float32)], axis=0)

    _agg = _sc_agg()
    ea0, ea1 = _agg(aug, ea2, dst2, zeros128)
    xs2b, _ = lax.optimization_barrier((xs2, ea0))
    p0, p1 = _agg(embed, xs2b, dst2, zeros128)
    for l in range(3):
        if l < 2:
            h = _boundary(p0, p1, ea0, ea1, W_node[l], W_edge[l], b[l][None])
            p0, p1 = _agg(h, src2, dst2, zeros128)
        else:
            out = _final(p0, p1, ea0, ea1, W_node[2], W_edge[2], b[2][None], oh)
    return out


# speedup vs baseline: 1.0477x; 1.0477x over previous
import functools

import jax
import jax.numpy as jnp
from jax import lax
from jax.experimental import pallas as pl
from jax.experimental.pallas import tpu as pltpu
from jax.experimental.pallas import tpu_sc as plsc

N_NODES = 10000
N_PAD = 10240
E = 320000
D = 128
NG = 64
NC = 2
NS = 16
NW = NC * NS
CHUNK = 128
EPT = 80
E_PAD = NW * EPT * CHUNK
NCHUNKS = E_PAD // CHUNK
NBUF = 4
RPT = N_PAD // NS
BLK = 512
GRID = N_PAD // BLK


@functools.lru_cache(maxsize=None)
def _sc_agg():
    mesh = plsc.VectorSubcoreMesh(
        core_axis_name="c", subcore_axis_name="s",
        num_cores=NC, num_subcores=NS)

    @functools.partial(
        pl.kernel,
        out_type=(
            jax.ShapeDtypeStruct((N_PAD, D), jnp.float32),
            jax.ShapeDtypeStruct((N_PAD, D), jnp.float32),
        ),
        mesh=mesh,
        scratch_types=[
            pltpu.VMEM((EPT, CHUNK), jnp.int32),
            pltpu.VMEM((EPT, CHUNK), jnp.int32),
            [pltpu.VMEM((CHUNK, D), jnp.float32)] * NBUF,
            [pltpu.SemaphoreType.DMA] * NBUF,
            [pltpu.SemaphoreType.DMA] * NBUF,
            pltpu.VMEM_SHARED((N_PAD, D), jnp.float32),
        ],
    )
    def _agg(table, src2, dst2, zeros, out0, out1,
             src_v, dst_v, rows, gsem, ssem, acc):
        c = lax.axis_index("c")
        s = lax.axis_index("s")
        r0 = pl.multiple_of(s * RPT, RPT)
        pltpu.sync_copy(zeros, acc.at[pl.ds(r0, RPT)])
        wid = c * NS + s
        cbase = pl.multiple_of(wid * EPT, EPT)
        pltpu.sync_copy(src2.at[pl.ds(cbase, EPT)], src_v)
        pltpu.sync_copy(dst2.at[pl.ds(cbase, EPT)], dst_v)
        plsc.subcore_barrier()

        def body(g, carry):
            pltpu.async_copy(table.at[src_v.at[g]], rows[0], gsem[0]).wait()
            pltpu.sync_copy(rows[0], acc.at[dst_v.at[g]], add=True)
            return carry

        lax.fori_loop(0, EPT, body, 0)
        plsc.subcore_barrier()

        @pl.when(c == 0)
        def _():
            pltpu.sync_copy(acc.at[pl.ds(r0, RPT)], out0.at[pl.ds(r0, RPT)])

        @pl.when(c == 1)
        def _():
            pltpu.sync_copy(acc.at[pl.ds(r0, RPT)], out1.at[pl.ds(r0, RPT)])

    return _agg


def _hidden(p0, p1, ea0, ea1, wn, we, bias):
    agg = jnp.dot(p0[...] + p1[...], wn[...],
                  preferred_element_type=jnp.float32)
    e_blk = ea0[...] + ea1[...]
    esum = e_blk[:, :32]
    deg = e_blk[:, 32:33]
    aggf = agg + jnp.dot(esum, we[...], preferred_element_type=jnp.float32)
    return jnp.maximum(aggf / jnp.maximum(deg, 1.0) + bias[...], 0.0)


def _boundary_body(p0, p1, ea0, ea1, wn, we, bias, o_ref):
    o_ref[...] = _hidden(p0, p1, ea0, ea1, wn, we, bias)


_NODE_SPECS = [
    pl.BlockSpec((BLK, D), lambda i: (i, 0)),
    pl.BlockSpec((BLK, D), lambda i: (i, 0)),
    pl.BlockSpec((BLK, D), lambda i: (i, 0)),
    pl.BlockSpec((BLK, D), lambda i: (i, 0)),
    pl.BlockSpec((D, D), lambda i: (0, 0)),
    pl.BlockSpec((32, D), lambda i: (0, 0)),
    pl.BlockSpec((1, D), lambda i: (0, 0)),
]


def _boundary(p0, p1, ea0, ea1, wn, we, bias):
    return pl.pallas_call(
        _boundary_body,
        out_shape=jax.ShapeDtypeStruct((N_PAD, D), jnp.float32),
        grid=(GRID,),
        in_specs=_NODE_SPECS,
        out_specs=pl.BlockSpec((BLK, D), lambda i: (i, 0)),
    )(p0, p1, ea0, ea1, wn, we, bias)


def _final_body(p0, p1, ea0, ea1, wn, we, bias, oh_ref, o_ref):
    h = _hidden(p0, p1, ea0, ea1, wn, we, bias)

    @pl.when(pl.program_id(0) == 0)
    def _():
        o_ref[...] = jnp.zeros_like(o_ref)

    oh = oh_ref[...]
    rows = [jnp.max(h * oh[:, g:g + 1], axis=0) for g in range(NG)]
    o_ref[...] = jnp.maximum(o_ref[...], jnp.stack(rows, axis=0))


def _final(p0, p1, ea0, ea1, wn, we, bias, oh):
    return pl.pallas_call(
        _final_body,
        out_shape=jax.ShapeDtypeStruct((NG, D), jnp.float32),
        grid=(GRID,),
        in_specs=_NODE_SPECS + [pl.BlockSpec((BLK, NG), lambda i: (i, 0))],
        out_specs=pl.BlockSpec((NG, D), lambda i: (0, 0)),
    )(p0, p1, ea0, ea1, wn, we, bias, oh)


def kernel(x, edge_attr, edge_index, batch, embed, edge_embed, W_node, W_edge, b):
    x = x.astype(jnp.int32)
    ea = edge_attr.astype(jnp.int32)
    src = edge_index[0].astype(jnp.int32)
    dst = edge_index[1].astype(jnp.int32)
    bt = batch.astype(jnp.int32)

    pad_e = E_PAD - E
    xs = x[src]
    xs_pad = jnp.concatenate([xs, jnp.zeros((pad_e,), jnp.int32)])
    src_pad = jnp.concatenate([src, jnp.zeros((pad_e,), jnp.int32)])
    trash = N_NODES + (jnp.arange(pad_e, dtype=jnp.int32) % (N_PAD - N_NODES))
    dst_pad = jnp.concatenate([dst, trash])
    ea_pad = jnp.concatenate([ea, jnp.full((pad_e,), 200, jnp.int32)])
    xs2 = xs_pad.reshape(NCHUNKS, CHUNK)
    src2 = src_pad.reshape(NCHUNKS, CHUNK)
    dst2 = dst_pad.reshape(NCHUNKS, CHUNK)
    ea2 = ea_pad.reshape(NCHUNKS, CHUNK)

    aug = jnp.zeros((208, D), jnp.float32)
    aug = aug.at[:200, :32].set(edge_embed).at[:200, 32].set(1.0)

    zeros128 = jnp.zeros((RPT, D), jnp.float32)
    oh = jnp.concatenate(
        [jax.nn.one_hot(bt, NG, dtype=jnp.float32),
         jnp.zeros((N_PAD - N_NODES, NG), jnp.float32)], axis=0)

    _agg = _sc_agg()
    ea0, ea1 = _agg(aug, ea2, dst2, zeros128)
    xs2b, _ = lax.optimization_barrier((xs2, ea0))
    p0, p1 = _agg(embed, xs2b, dst2, zeros128)
    for l in range(3):
        if l < 2:
            h = _boundary(p0, p1, ea0, ea1, W_node[l], W_edge[l], b[l][None])
            p0, p1 = _agg(h, src2, dst2, zeros128)
        else:
            out = _final(p0, p1, ea0, ea1, W_node[2], W_edge[2], b[2][None], oh)
    return out


# speedup vs baseline: 1.9265x; 1.8387x over previous
import functools

import jax
import jax.numpy as jnp
from jax import lax
from jax.experimental import pallas as pl
from jax.experimental.pallas import tpu as pltpu
from jax.experimental.pallas import tpu_sc as plsc

N_NODES = 10000
N_PAD = 10240
E = 320000
D = 128
AUG = 128
NG = 64
NC = 2
NS = 16
NW = NC * NS
CHUNK = 128
EPT = 79
E_PAD = NW * EPT * CHUNK
X_PAD = 12288
RPT = N_PAD // NS
BLK = 512
GRID = N_PAD // BLK


def _wid():
    return lax.axis_index("c") * NS + lax.axis_index("s")


@functools.lru_cache(maxsize=None)
def _sc_kernels():
    mesh = plsc.VectorSubcoreMesh(
        core_axis_name="c", subcore_axis_name="s",
        num_cores=NC, num_subcores=NS)

    @functools.partial(
        pl.kernel,
        out_type=jax.ShapeDtypeStruct((X_PAD, D), jnp.float32),
        mesh=mesh,
        scratch_types=[
            pltpu.VMEM((CHUNK,), jnp.int32),
            pltpu.VMEM((CHUNK, D), jnp.float32),
            pltpu.SemaphoreType.DMA,
        ],
    )
    def _gather(table, idx, out, idx_v, rows_v, sem):
        base = _wid() * ((X_PAD // NW // CHUNK) * CHUNK)

        def body(j, carry):
            off = pl.multiple_of(base + j * CHUNK, CHUNK)
            pltpu.sync_copy(idx.at[pl.ds(off, CHUNK)], idx_v)
            pltpu.async_copy(table.at[idx_v], rows_v, sem).wait()
            pltpu.sync_copy(rows_v, out.at[pl.ds(off, CHUNK)])
            return carry

        lax.fori_loop(0, X_PAD // NW // CHUNK, body, 0)

    @functools.partial(
        pl.kernel,
        out_type=(
            jax.ShapeDtypeStruct((N_PAD, D), jnp.float32),
            jax.ShapeDtypeStruct((N_PAD, D), jnp.float32),
        ),
        mesh=mesh,
        scratch_types=[
            pltpu.VMEM((CHUNK,), jnp.int32),
            pltpu.VMEM((CHUNK,), jnp.int32),
            pltpu.VMEM((CHUNK, D), jnp.float32),
            pltpu.VMEM_SHARED((N_PAD, D), jnp.float32),
            pltpu.SemaphoreType.DMA,
        ],
    )
    def _agg(table, src, dst, zeros, out0, out1,
             src_v, dst_v, rows_v, acc, sem):
        c = lax.axis_index("c")
        s = lax.axis_index("s")
        r0 = pl.multiple_of(s * RPT, RPT)
        pltpu.sync_copy(zeros.at[pl.ds(r0, RPT)], acc.at[pl.ds(r0, RPT)])
        plsc.subcore_barrier()
        base = _wid() * (EPT * CHUNK)

        def body(j, carry):
            off = pl.multiple_of(base + j * CHUNK, CHUNK)
            pltpu.sync_copy(src.at[pl.ds(off, CHUNK)], src_v)
            pltpu.sync_copy(dst.at[pl.ds(off, CHUNK)], dst_v)
            pltpu.async_copy(table.at[src_v], rows_v, sem).wait()
            pltpu.sync_copy(rows_v, acc.at[dst_v], add=True)
            return carry

        lax.fori_loop(0, EPT, body, 0)
        plsc.subcore_barrier()

        @pl.when(c == 0)
        def _():
            pltpu.sync_copy(acc.at[pl.ds(r0, RPT)], out0.at[pl.ds(r0, RPT)])

        @pl.when(c == 1)
        def _():
            pltpu.sync_copy(acc.at[pl.ds(r0, RPT)], out1.at[pl.ds(r0, RPT)])

    return _gather, _agg


def _mm_body(h_ref, w_ref, o_ref):
    o_ref[...] = jnp.dot(h_ref[...], w_ref[...],
                         preferred_element_type=jnp.float32)


def _mm(h, w):
    return pl.pallas_call(
        _mm_body,
        out_shape=jax.ShapeDtypeStruct((N_PAD, D), jnp.float32),
        grid=(GRID,),
        in_specs=[
            pl.BlockSpec((BLK, D), lambda i: (i, 0)),
            pl.BlockSpec((D, D), lambda i: (0, 0)),
        ],
        out_specs=pl.BlockSpec((BLK, D), lambda i: (i, 0)),
    )(h, w)


def _hidden(p0, p1, ea0, ea1, we, bias):
    agg = p0[...] + p1[...]
    e_blk = ea0[...] + ea1[...]
    esum = e_blk[:, :32]
    deg = e_blk[:, 32:33]
    aggf = agg + jnp.dot(esum, we[...], preferred_element_type=jnp.float32)
    return jnp.maximum(aggf / jnp.maximum(deg, 1.0) + bias[...], 0.0)


def _boundary_body(p0, p1, ea0, ea1, we, bias, wn, o_ref):
    h = _hidden(p0, p1, ea0, ea1, we, bias)
    o_ref[...] = jnp.dot(h, wn[...], preferred_element_type=jnp.float32)


def _boundary(p0, p1, ea0, ea1, we, bias, wn):
    return pl.pallas_call(
        _boundary_body,
        out_shape=jax.ShapeDtypeStruct((N_PAD, D), jnp.float32),
        grid=(GRID,),
        in_specs=[
            pl.BlockSpec((BLK, D), lambda i: (i, 0)),
            pl.BlockSpec((BLK, D), lambda i: (i, 0)),
            pl.BlockSpec((BLK, AUG), lambda i: (i, 0)),
            pl.BlockSpec((BLK, AUG), lambda i: (i, 0)),
            pl.BlockSpec((32, D), lambda i: (0, 0)),
            pl.BlockSpec((1, D), lambda i: (0, 0)),
            pl.BlockSpec((D, D), lambda i: (0, 0)),
        ],
        out_specs=pl.BlockSpec((BLK, D), lambda i: (i, 0)),
    )(p0, p1, ea0, ea1, we, bias, wn)


def _final_body(p0, p1, ea0, ea1, we, bias, oh_ref, o_ref):
    h = _hidden(p0, p1, ea0, ea1, we, bias)

    @pl.when(pl.program_id(0) == 0)
    def _():
        o_ref[...] = jnp.zeros_like(o_ref)

    oh = oh_ref[...]
    rows = [jnp.max(h * oh[:, g:g + 1], axis=0) for g in range(NG)]
    o_ref[...] = jnp.maximum(o_ref[...], jnp.stack(rows, axis=0))


def _final(p0, p1, ea0, ea1, we, bias, oh):
    return pl.pallas_call(
        _final_body,
        out_shape=jax.ShapeDtypeStruct((NG, D), jnp.float32),
        grid=(GRID,),
        in_specs=[
            pl.BlockSpec((BLK, D), lambda i: (i, 0)),
            pl.BlockSpec((BLK, D), lambda i: (i, 0)),
            pl.BlockSpec((BLK, AUG), lambda i: (i, 0)),
            pl.BlockSpec((BLK, AUG), lambda i: (i, 0)),
            pl.BlockSpec((32, D), lambda i: (0, 0)),
            pl.BlockSpec((1, D), lambda i: (0, 0)),
            pl.BlockSpec((BLK, NG), lambda i: (i, 0)),
        ],
        out_specs=pl.BlockSpec((NG, D), lambda i: (0, 0)),
    )(p0, p1, ea0, ea1, we, bias, oh)


def kernel(x, edge_attr, edge_index, batch, embed, edge_embed, W_node, W_edge, b):
    x = x.astype(jnp.int32)
    ea = edge_attr.astype(jnp.int32)
    src = edge_index[0].astype(jnp.int32)
    dst = edge_index[1].astype(jnp.int32)
    bt = batch.astype(jnp.int32)

    pad_e = E_PAD - E
    x_pad = jnp.concatenate([x, jnp.zeros((X_PAD - N_NODES,), jnp.int32)])
    src_pad = jnp.concatenate([src, jnp.zeros((pad_e,), jnp.int32)])
    dst_pad = jnp.concatenate([dst, jnp.full((pad_e,), N_NODES, jnp.int32)])
    ea_pad = jnp.concatenate([ea, jnp.full((pad_e,), 200, jnp.int32)])

    aug = jnp.zeros((208, AUG), jnp.float32)
    aug = aug.at[:200, :32].set(edge_embed).at[:200, 32].set(1.0)

    zeros128 = jnp.zeros((N_PAD, D), jnp.float32)
    oh = jnp.concatenate(
        [jax.nn.one_hot(bt, NG, dtype=jnp.float32),
         jnp.zeros((N_PAD - N_NODES, NG), jnp.float32)], axis=0)

    _gather, _agg = _sc_kernels()
    h0 = _gather(embed, x_pad)
    ea0, ea1 = _agg(aug, ea_pad, dst_pad, zeros128)
    hw = _mm(h0, W_node[0])
    for l in range(3):
        p0, p1 = _agg(hw, src_pad, dst_pad, zeros128)
        if l < 2:
            hw = _boundary(p0, p1, ea0, ea1, W_edge[l], b[l][None], W_node[l + 1])
        else:
            out = _final(p0, p1, ea0, ea1, W_edge[2], b[2][None], oh)
    return out


# speedup vs baseline: 2.0130x; 1.0449x over previous
import functools

import jax
import jax.numpy as jnp
from jax import lax
from jax.experimental import pallas as pl
from jax.experimental.pallas import tpu as pltpu
from jax.experimental.pallas import tpu_sc as plsc

N_NODES = 10000
N_PAD = 10240
E = 320000
D = 128
AUG = 128
NG = 64
NC = 2
NS = 16
NW = NC * NS
CHUNK = 128
EPT = 79
E_PAD = NW * EPT * CHUNK
X_PAD = 12288
RPT = N_PAD // NS
BLK = 512
GRID = N_PAD // BLK


def _wid():
    return lax.axis_index("c") * NS + lax.axis_index("s")


@functools.lru_cache(maxsize=None)
def _sc_kernels():
    mesh = plsc.VectorSubcoreMesh(
        core_axis_name="c", subcore_axis_name="s",
        num_cores=NC, num_subcores=NS)

    @functools.partial(
        pl.kernel,
        out_type=jax.ShapeDtypeStruct((X_PAD, D), jnp.float32),
        mesh=mesh,
        scratch_types=[
            pltpu.VMEM((CHUNK,), jnp.int32),
            pltpu.VMEM((CHUNK, D), jnp.float32),
            pltpu.SemaphoreType.DMA,
        ],
    )
    def _gather(table, idx, out, idx_v, rows_v, sem):
        base = _wid() * ((X_PAD // NW // CHUNK) * CHUNK)

        def body(j, carry):
            off = pl.multiple_of(base + j * CHUNK, CHUNK)
            pltpu.sync_copy(idx.at[pl.ds(off, CHUNK)], idx_v)
            pltpu.async_copy(table.at[idx_v], rows_v, sem).wait()
            pltpu.sync_copy(rows_v, out.at[pl.ds(off, CHUNK)])
            return carry

        lax.fori_loop(0, X_PAD // NW // CHUNK, body, 0)

    @functools.partial(
        pl.kernel,
        out_type=(
            jax.ShapeDtypeStruct((N_PAD, D), jnp.float32),
            jax.ShapeDtypeStruct((N_PAD, D), jnp.float32),
        ),
        mesh=mesh,
        scratch_types=[
            pltpu.VMEM((CHUNK,), jnp.int32),
            pltpu.VMEM((CHUNK,), jnp.int32),
            pltpu.VMEM((CHUNK, D), jnp.float32),
            pltpu.VMEM_SHARED((N_PAD, D), jnp.float32),
            pltpu.SemaphoreType.DMA,
        ],
    )
    def _agg(table, src, dst, zeros, out0, out1,
             src_v, dst_v, rows_v, acc, sem):
        c = lax.axis_index("c")
        s = lax.axis_index("s")
        r0 = pl.multiple_of(s * RPT, RPT)
        pltpu.sync_copy(zeros.at[pl.ds(r0, RPT)], acc.at[pl.ds(r0, RPT)])
        plsc.subcore_barrier()
        base = _wid() * (EPT * CHUNK)

        def body(j, carry):
            off = pl.multiple_of(base + j * CHUNK, CHUNK)
            pltpu.sync_copy(src.at[pl.ds(off, CHUNK)], src_v)
            pltpu.sync_copy(dst.at[pl.ds(off, CHUNK)], dst_v)
            pltpu.async_copy(table.at[src_v], rows_v, sem).wait()
            pltpu.sync_copy(rows_v, acc.at[dst_v], add=True)
            return carry

        lax.fori_loop(0, EPT, body, 0)
        plsc.subcore_barrier()

        @pl.when(c == 0)
        def _():
            pltpu.sync_copy(acc.at[pl.ds(r0, RPT)], out0.at[pl.ds(r0, RPT)])

        @pl.when(c == 1)
        def _():
            pltpu.sync_copy(acc.at[pl.ds(r0, RPT)], out1.at[pl.ds(r0, RPT)])

    return _gather, _agg


def _mm_body(h_ref, w_ref, o_ref):
    o_ref[...] = jnp.dot(h_ref[...], w_ref[...],
                         preferred_element_type=jnp.float32)


def _mm(h, w):
    return pl.pallas_call(
        _mm_body,
        out_shape=jax.ShapeDtypeStruct((N_PAD, D), jnp.float32),
        grid=(GRID,),
        in_specs=[
            pl.BlockSpec((BLK, D), lambda i: (i, 0)),
            pl.BlockSpec((D, D), lambda i: (0, 0)),
        ],
        out_specs=pl.BlockSpec((BLK, D), lambda i: (i, 0)),
    )(h, w)


def _hidden(p0, p1, ea0, ea1, wn, we, bias):
    agg = jnp.dot(p0[...] + p1[...], wn[...],
                  preferred_element_type=jnp.float32)
    e_blk = ea0[...] + ea1[...]
    esum = e_blk[:, :32]
    deg = e_blk[:, 32:33]
    aggf = agg + jnp.dot(esum, we[...], preferred_element_type=jnp.float32)
    return jnp.maximum(aggf / jnp.maximum(deg, 1.0) + bias[...], 0.0)


def _boundary_body(p0, p1, ea0, ea1, wn, we, bias, o_ref):
    o_ref[...] = _hidden(p0, p1, ea0, ea1, wn, we, bias)


def _boundary(p0, p1, ea0, ea1, wn, we, bias):
    return pl.pallas_call(
        _boundary_body,
        out_shape=jax.ShapeDtypeStruct((N_PAD, D), jnp.float32),
        grid=(GRID,),
        in_specs=[
            pl.BlockSpec((BLK, D), lambda i: (i, 0)),
            pl.BlockSpec((BLK, D), lambda i: (i, 0)),
            pl.BlockSpec((BLK, AUG), lambda i: (i, 0)),
            pl.BlockSpec((BLK, AUG), lambda i: (i, 0)),
            pl.BlockSpec((D, D), lambda i: (0, 0)),
            pl.BlockSpec((32, D), lambda i: (0, 0)),
            pl.BlockSpec((1, D), lambda i: (0, 0)),
        ],
        out_specs=pl.BlockSpec((BLK, D), lambda i: (i, 0)),
    )(p0, p1, ea0, ea1, wn, we, bias)


def _final_body(p0, p1, ea0, ea1, wn, we, bias, oh_ref, o_ref):
    h = _hidden(p0, p1, ea0, ea1, wn, we, bias)

    @pl.when(pl.program_id(0) == 0)
    def _():
        o_ref[...] = jnp.zeros_like(o_ref)

    oh = oh_ref[...]
    rows = [jnp.max(h * oh[:, g:g + 1], axis=0) for g in range(NG)]
    o_ref[...] = jnp.maximum(o_ref[...], jnp.stack(rows, axis=0))


def _final(p0, p1, ea0, ea1, wn, we, bias, oh):
    return pl.pallas_call(
        _final_body,
        out_shape=jax.ShapeDtypeStruct((NG, D), jnp.float32),
        grid=(GRID,),
        in_specs=[
            pl.BlockSpec((BLK, D), lambda i: (i, 0)),
            pl.BlockSpec((BLK, D), lambda i: (i, 0)),
            pl.BlockSpec((BLK, AUG), lambda i: (i, 0)),
            pl.BlockSpec((BLK, AUG), lambda i: (i, 0)),
            pl.BlockSpec((D, D), lambda i: (0, 0)),
            pl.BlockSpec((32, D), lambda i: (0, 0)),
            pl.BlockSpec((1, D), lambda i: (0, 0)),
            pl.BlockSpec((BLK, NG), lambda i: (i, 0)),
        ],
        out_specs=pl.BlockSpec((NG, D), lambda i: (0, 0)),
    )(p0, p1, ea0, ea1, wn, we, bias, oh)


def kernel(x, edge_attr, edge_index, batch, embed, edge_embed, W_node, W_edge, b):
    x = x.astype(jnp.int32)
    ea = edge_attr.astype(jnp.int32)
    src = edge_index[0].astype(jnp.int32)
    dst = edge_index[1].astype(jnp.int32)
    bt = batch.astype(jnp.int32)

    pad_e = E_PAD - E
    x_pad = jnp.concatenate([x, jnp.zeros((X_PAD - N_NODES,), jnp.int32)])
    src_pad = jnp.concatenate([src, jnp.zeros((pad_e,), jnp.int32)])
    dst_pad = jnp.concatenate([dst, jnp.full((pad_e,), N_NODES, jnp.int32)])
    ea_pad = jnp.concatenate([ea, jnp.full((pad_e,), 200, jnp.int32)])

    aug = jnp.zeros((208, AUG), jnp.float32)
    aug = aug.at[:200, :32].set(edge_embed).at[:200, 32].set(1.0)

    zeros128 = jnp.zeros((N_PAD, D), jnp.float32)
    oh = jnp.concatenate(
        [jax.nn.one_hot(bt, NG, dtype=jnp.float32),
         jnp.zeros((N_PAD - N_NODES, NG), jnp.float32)], axis=0)

    _gather, _agg = _sc_kernels()
    h0 = _gather(embed, x_pad)
    ea0, ea1 = _agg(aug, ea_pad, dst_pad, zeros128)
    h = h0
    for l in range(3):
        p0, p1 = _agg(h, src_pad, dst_pad, zeros128)
        if l < 2:
            h = _boundary(p0, p1, ea0, ea1, W_node[l], W_edge[l], b[l][None])
        else:
            out = _final(p0, p1, ea0, ea1, W_node[2], W_edge[2], b[2][None], oh)
    return out
